# baseline (speedup 1.0000x reference)
import jax
import jax.numpy as jnp
from jax import lax
from jax.experimental import pallas as pl
from jax.experimental.pallas import tpu as pltpu
from jax.experimental.pallas import tpu_sc as plsc

B, L, D = 4096, 200, 64
LP = 208
EPS = 1e-12
NC, NS = 2, 16
NW = NC * NS
RPW = B // NW
C0, C1 = 104, 96
NACC = 8


def _rsqrt(x):
    i = lax.bitcast_convert_type(x, jnp.int32)
    i = jnp.int32(0x5F3759DF) - (i >> 1)
    y = lax.bitcast_convert_type(i, jnp.float32)
    for _ in range(3):
        y = y * (1.5 - 0.5 * x * y * y)
    return y


def _body(word_hbm, pos_hbm, cat_hbm, gamma_hbm, beta_hbm, tok_hbm, catidx_hbm,
          out_hbm,
          pos_v, idx_v, catid_v, catrow_v, gb_v, words_v, out_v,
          gsem0, gsem1, osem0, osem1):
    wid = lax.axis_index("s") * NC + lax.axis_index("c")
    base = wid * RPW
    gsems = (gsem0, gsem1)
    osems = (osem0, osem1)

    pltpu.sync_copy(pos_hbm.at[pl.ds(0, L)], pos_v.at[pl.ds(0, L)])
    pltpu.sync_copy(gamma_hbm, gb_v.at[pl.ds(0, D)])
    pltpu.sync_copy(beta_hbm, gb_v.at[pl.ds(D, D)])
    pltpu.sync_copy(tok_hbm.at[pl.ds(base * L, RPW * L)], idx_v)
    pltpu.sync_copy(catidx_hbm.at[pl.ds(base, RPW)], catid_v)
    pltpu.async_copy(cat_hbm.at[catid_v], catrow_v, gsem0).wait()

    def gather_copies(r, b):
        roff = pl.multiple_of(r * L, 8)
        cp0 = pltpu.make_async_copy(
            word_hbm.at[idx_v.at[pl.ds(roff, C0)]],
            words_v.at[pl.ds(b * LP, C0)], gsems[b])
        cp1 = pltpu.make_async_copy(
            word_hbm.at[idx_v.at[pl.ds(pl.multiple_of(roff + C0, 8), C1)]],
            words_v.at[pl.ds(b * LP + C0, C1)], gsems[b])
        return cp0, cp1

    lane = lax.iota(jnp.int32, 16)

    def compute_row(r, b):
        bbase = b * LP
        splat_r = jnp.broadcast_to(r, (16,)).astype(jnp.int32)

        def group_body(g, carry):
            rowidx = g * 16 + lane
            gr = bbase + rowidx
            accs = [None] * NACC
            sqs = [None] * NACC
            for d in range(D):
                cold = jnp.full((16,), d, jnp.int32)
                w = plsc.load_gather(words_v, [gr, cold])
                p = plsc.load_gather(pos_v, [rowidx, cold])
                c = plsc.load_gather(catrow_v, [splat_r, cold])
                x = (w + p) + c
                plsc.store_scatter(out_v, [gr, cold], x)
                a = d % NACC
                accs[a] = x if accs[a] is None else accs[a] + x
                sqs[a] = x * x if sqs[a] is None else sqs[a] + x * x
            while len(accs) > 1:
                accs = [accs[i] + accs[i + 1] for i in range(0, len(accs), 2)]
                sqs = [sqs[i] + sqs[i + 1] for i in range(0, len(sqs), 2)]
            mean = accs[0] * (1.0 / D)
            var = sqs[0] * (1.0 / D) - mean * mean
            inv = _rsqrt(var + EPS)
            minv = mean * inv
            for d in range(D):
                cold = jnp.full((16,), d, jnp.int32)
                x = plsc.load_gather(out_v, [gr, cold])
                g_s = plsc.load_gather(gb_v, [cold])
                b_s = plsc.load_gather(gb_v, [jnp.full((16,), D + d, jnp.int32)])
                y = (x * inv - minv) * g_s + b_s
                plsc.store_scatter(out_v, [gr, cold], y)
            return carry

        lax.fori_loop(0, LP // 16, group_body, 0)

    for b in range(2):
        cp0, cp1 = gather_copies(b, b)
        cp0.start()
        cp1.start()

    def row_body(i, carry):
        for b in range(2):
            r = i * 2 + b
            cp0, cp1 = gather_copies(r, b)
            cp0.wait()
            cp1.wait()

            @pl.when(r >= 2)
            def _():
                pltpu.make_async_copy(
                    out_v.at[pl.ds(b * LP, L)],
                    out_hbm.at[base + r - 2], osems[b]).wait()

            compute_row(r, b)
            pltpu.make_async_copy(
                out_v.at[pl.ds(b * LP, L)],
                out_hbm.at[base + r], osems[b]).start()

            @pl.when(r + 2 < RPW)
            def _():
                n0, n1 = gather_copies(r + 2, b)
                n0.start()
                n1.start()
        return carry

    lax.fori_loop(0, RPW // 2, row_body, 0)
    pltpu.make_async_copy(
        out_v.at[pl.ds(0, L)], out_hbm.at[base + RPW - 2], osem0).wait()
    pltpu.make_async_copy(
        out_v.at[pl.ds(LP, L)], out_hbm.at[base + RPW - 1], osem1).wait()


_emb_ln = pl.kernel(
    _body,
    out_type=jax.ShapeDtypeStruct((B, L, D), jnp.float32),
    mesh=plsc.VectorSubcoreMesh(core_axis_name="c", subcore_axis_name="s"),
    compiler_params=pltpu.CompilerParams(
        needs_layout_passes=False, use_tc_tiling_on_sc=False),
    scratch_types=[
        pltpu.VMEM((LP, D), jnp.float32),
        pltpu.VMEM((RPW * L,), jnp.int32),
        pltpu.VMEM((RPW,), jnp.int32),
        pltpu.VMEM((RPW, D), jnp.float32),
        pltpu.VMEM((2 * D,), jnp.float32),
        pltpu.VMEM((2 * LP, D), jnp.float32),
        pltpu.VMEM((2 * LP, D), jnp.float32),
        pltpu.SemaphoreType.DMA,
        pltpu.SemaphoreType.DMA,
        pltpu.SemaphoreType.DMA,
        pltpu.SemaphoreType.DMA,
    ],
)


def kernel(word_emb, pos_emb, cat_emb, gamma, beta, tag_tokens, category):
    tok = tag_tokens.reshape(-1).astype(jnp.int32)
    cat = category.reshape(-1).astype(jnp.int32)
    return _emb_ln(word_emb, pos_emb, cat_emb, gamma, beta, tok, cat)


# speedup vs baseline: 4.6292x; 4.6292x over previous
import jax
import jax.numpy as jnp
from jax import lax
from jax.experimental import pallas as pl
from jax.experimental.pallas import tpu as pltpu
from jax.experimental.pallas import tpu_sc as plsc

B, L, D = 4096, 200, 64
LP = 208
NG = 13
EPS = 1e-12
NC, NS = 2, 16
NW = NC * NS
RPW = B // NW
C0, C1 = 104, 96
GSPLIT = (range(0, 7), range(7, 13))


def _rsqrt(x):
    i = lax.bitcast_convert_type(x, jnp.int32)
    i = jnp.int32(0x5F3759DF) - (i >> 1)
    y = lax.bitcast_convert_type(i, jnp.float32)
    for _ in range(3):
        y = y * (1.5 - 0.5 * x * y * y)
    return y


def _body(word_hbm, pos_hbm, cat_hbm, gamma_hbm, beta_hbm, tok_hbm, catidx_hbm,
          out_hbm,
          pos_v, idx_v, catid_v, catrow_v, gb_v, stats_v, words_v, out_v,
          gsem0, gsem1, osem0, osem1):
    wid = lax.axis_index("s") * NC + lax.axis_index("c")
    base = wid * RPW
    gsems = (gsem0, gsem1)
    osems = (osem0, osem1)

    pltpu.sync_copy(pos_hbm.at[pl.ds(0, L)], pos_v.at[pl.ds(0, L)])
    pltpu.sync_copy(gamma_hbm, gb_v.at[pl.ds(0, D)])
    pltpu.sync_copy(beta_hbm, gb_v.at[pl.ds(D, D)])
    pltpu.sync_copy(tok_hbm.at[pl.ds(base * L, RPW * L)], idx_v)
    pltpu.sync_copy(catidx_hbm.at[pl.ds(base, RPW)], catid_v)
    pltpu.async_copy(cat_hbm.at[catid_v], catrow_v, gsem0).wait()

    lane = lax.iota(jnp.int32, 16)

    def gather_copies(r, b):
        roff = pl.multiple_of(r * L, 8)
        cp0 = pltpu.make_async_copy(
            word_hbm.at[idx_v.at[pl.ds(roff, C0)]],
            words_v.at[pl.ds(b * LP, C0)], gsems[b])
        cp1 = pltpu.make_async_copy(
            word_hbm.at[idx_v.at[pl.ds(pl.multiple_of(roff + C0, 8), C1)]],
            words_v.at[pl.ds(b * LP + C0, C1)], gsems[b])
        return cp0, cp1

    def prep_row(r, b):
        rsplat = jnp.broadcast_to(r, (16,)).astype(jnp.int32)
        cq = [plsc.load_gather(catrow_v, [rsplat, i * 16 + lane])
              for i in range(4)]

        @plsc.parallel_loop(0, L, unroll=4)
        def _tok_body(t):
            psplat = jnp.broadcast_to(t, (16,)).astype(jnp.int32)
            wsplat = psplat + b * LP
            for i in range(4):
                col = i * 16 + lane
                p = plsc.load_gather(pos_v, [psplat, col])
                plsc.store_scatter(words_v, [wsplat, col], p + cq[i])

        cp0, cp1 = gather_copies(r, b)
        cp0.start(add=True)
        cp1.start(add=True)

    def compute_row(b):
        bbase = b * LP
        grs = [jnp.broadcast_to(bbase + g * 16, (16,)).astype(jnp.int32) + lane
               for g in range(NG)]

        for groups in GSPLIT:
            z = jnp.zeros((16,), jnp.float32)
            n = len(groups)

            @plsc.parallel_loop(0, D, unroll=4, carry=((z,) * n, (z,) * n))
            def _p1_body(d, carry):
                skd = (jnp.broadcast_to(d, (16,)).astype(jnp.int32)
                       + lane) & (D - 1)
                accs, sqs = carry
                accs = list(accs)
                sqs = list(sqs)
                for k, g in enumerate(groups):
                    x = plsc.load_gather(words_v, [grs[g], skd])
                    accs[k] = accs[k] + x
                    sqs[k] = sqs[k] + x * x
                return tuple(accs), tuple(sqs)

            accs, sqs = _p1_body
            for k, g in enumerate(groups):
                mean = accs[k] * (1.0 / D)
                var = sqs[k] * (1.0 / D) - mean * mean
                inv = _rsqrt(var + EPS)
                stats_v[pl.ds(g * 32, 16)] = inv
                stats_v[pl.ds(g * 32 + 16, 16)] = mean * inv

        for groups in GSPLIT:
            invs = [stats_v[pl.ds(g * 32, 16)] for g in groups]
            minvs = [stats_v[pl.ds(g * 32 + 16, 16)] for g in groups]

            @plsc.parallel_loop(0, D, unroll=4)
            def _p2_body(d):
                skd = (jnp.broadcast_to(d, (16,)).astype(jnp.int32)
                       + lane) & (D - 1)
                gs = plsc.load_gather(gb_v, [skd])
                bs = plsc.load_gather(gb_v, [skd + D])
                for k, g in enumerate(groups):
                    x = plsc.load_gather(words_v, [grs[g], skd])
                    y = (x * invs[k] - minvs[k]) * gs + bs
                    plsc.store_scatter(out_v, [grs[g], skd], y)

    prep_row(0, 0)
    prep_row(1, 1)

    def row_body(i, carry):
        for b in range(2):
            r = i * 2 + b
            cp0, cp1 = gather_copies(r, b)
            cp0.wait()
            cp1.wait()

            @pl.when(r >= 2)
            def _():
                pltpu.make_async_copy(
                    out_v.at[pl.ds(b * LP, L)],
                    out_hbm.at[base + r - 2], osems[b]).wait()

            compute_row(b)
            pltpu.make_async_copy(
                out_v.at[pl.ds(b * LP, L)],
                out_hbm.at[base + r], osems[b]).start()

            @pl.when(r + 2 < RPW)
            def _():
                prep_row(r + 2, b)
        return carry

    lax.fori_loop(0, RPW // 2, row_body, 0)
    pltpu.make_async_copy(
        out_v.at[pl.ds(0, L)], out_hbm.at[base + RPW - 2], osem0).wait()
    pltpu.make_async_copy(
        out_v.at[pl.ds(LP, L)], out_hbm.at[base + RPW - 1], osem1).wait()


_emb_ln = pl.kernel(
    _body,
    out_type=jax.ShapeDtypeStruct((B, L, D), jnp.float32),
    mesh=plsc.VectorSubcoreMesh(core_axis_name="c", subcore_axis_name="s"),
    compiler_params=pltpu.CompilerParams(
        needs_layout_passes=False, use_tc_tiling_on_sc=False),
    scratch_types=[
        pltpu.VMEM((LP, D), jnp.float32),
        pltpu.VMEM((RPW * L,), jnp.int32),
        pltpu.VMEM((RPW,), jnp.int32),
        pltpu.VMEM((RPW, D), jnp.float32),
        pltpu.VMEM((2 * D,), jnp.float32),
        pltpu.VMEM((NG * 32,), jnp.float32),
        pltpu.VMEM((2 * LP, D), jnp.float32),
        pltpu.VMEM((2 * LP, D), jnp.float32),
        pltpu.SemaphoreType.DMA,
        pltpu.SemaphoreType.DMA,
        pltpu.SemaphoreType.DMA,
        pltpu.SemaphoreType.DMA,
    ],
)


def kernel(word_emb, pos_emb, cat_emb, gamma, beta, tag_tokens, category):
    tok = tag_tokens.reshape(-1).astype(jnp.int32)
    cat = category.reshape(-1).astype(jnp.int32)
    return _emb_ln(word_emb, pos_emb, cat_emb, gamma, beta, tok, cat)
